# Initial kernel scaffold; baseline (speedup 1.0000x reference)
import jax
import jax.numpy as jnp
from jax.experimental import pallas as pl


def kernel(x, hyperedge_index, W1, b1, W2, b2):
    raise NotImplementedError("write your pallas kernel here")


# speedup vs baseline: 14.0854x; 14.0854x over previous
import functools

import jax
import jax.numpy as jnp
from jax import lax
from jax.experimental import pallas as pl
from jax.experimental.pallas import tpu as pltpu
from jax.experimental.pallas import tpu_sc as plsc

N_NODES = 100000
N_HEDGES = 100000
NNZ = 6400000

BATCH = 128
N_TILES = 32
ROWS_PER_TILE = 1563
ROWS = ROWS_PER_TILE * N_TILES
NNZP = ROWS * BATCH
N_DUMP = 96
NPAD = N_NODES + N_DUMP
ZR = NPAD // 16


def _sc_phase(vtab, gidx, sidx, ztab):
    mesh = plsc.VectorSubcoreMesh(core_axis_name="c", subcore_axis_name="s")

    @functools.partial(
        pl.kernel,
        out_type=jax.ShapeDtypeStruct((2, NPAD, 8), jnp.float32),
        mesh=mesh,
        scratch_types=[
            pltpu.VMEM((BATCH,), jnp.int32),
            pltpu.VMEM((BATCH,), jnp.int32),
            pltpu.VMEM((BATCH, 8), jnp.float32),
            pltpu.VMEM_SHARED((NPAD, 8), jnp.float32),
            pltpu.SemaphoreType.DMA,
        ],
        compiler_params=pltpu.CompilerParams(use_tc_tiling_on_sc=False),
    )
    def phase(vtab_hbm, gidx_hbm, sidx_hbm, ztab_hbm, out_hbm,
              gbuf, sbuf, rbuf, acc, sem):
        cid = lax.axis_index("c")
        sid = lax.axis_index("s")
        wid = sid * 2 + cid

        pltpu.sync_copy(ztab_hbm.at[pl.ds(sid * ZR, ZR), :],
                        acc.at[pl.ds(sid * ZR, ZR), :])
        plsc.subcore_barrier()

        base = wid * ROWS_PER_TILE

        def step(j, carry):
            r = (base + j) * BATCH
            pltpu.sync_copy(gidx_hbm.at[pl.ds(r, BATCH)], gbuf)
            pltpu.sync_copy(sidx_hbm.at[pl.ds(r, BATCH)], sbuf)
            pltpu.async_copy(vtab_hbm.at[gbuf], rbuf, sem).wait()
            pltpu.sync_copy(rbuf, acc.at[sbuf], add=True)
            return carry

        lax.fori_loop(0, ROWS_PER_TILE, step, 0)
        plsc.subcore_barrier()

        pltpu.sync_copy(acc.at[pl.ds(sid * ZR, ZR), :],
                        out_hbm.at[cid, pl.ds(sid * ZR, ZR), :])

    return phase(vtab, gidx, sidx, ztab)


TC_BLK = 3128
TC_GRID = NPAD // TC_BLK


def _tc_combine(parts):
    def body(p_ref, o_ref):
        p = p_ref[0] + p_ref[1]
        cnt = p[:, 3:4]
        o_ref[...] = p * jnp.where(cnt > 0, 1.0 / cnt, 0.0)

    return pl.pallas_call(
        body,
        grid=(TC_GRID,),
        in_specs=[pl.BlockSpec((2, TC_BLK, 8), lambda i: (0, i, 0))],
        out_specs=pl.BlockSpec((TC_BLK, 8), lambda i: (i, 0)),
        out_shape=jax.ShapeDtypeStruct((NPAD, 8), jnp.float32),
    )(parts)


def _tc_final(parts, W1, b1, W2, b2):
    def body(p_ref, w1_ref, b1_ref, w2_ref, b2_ref, o_ref):
        p = p_ref[0] + p_ref[1]
        cnt = p[:, 3:4]
        scale = jnp.where(cnt > 0, 1.0 / cnt, 0.0)
        z = p[:, :3] * scale
        d = jnp.where(cnt > 0, 1.0, 0.0)
        t = (jnp.dot(z, w1_ref[...], preferred_element_type=jnp.float32)
             + d * b1_ref[...][None, :])
        o_ref[...] = (jnp.dot(t, w2_ref[...], preferred_element_type=jnp.float32)
                      + b2_ref[...][None, :])

    full = pl.pallas_call(
        body,
        grid=(TC_GRID,),
        in_specs=[
            pl.BlockSpec((2, TC_BLK, 8), lambda i: (0, i, 0)),
            pl.BlockSpec((3, 16), lambda i: (0, 0)),
            pl.BlockSpec((16,), lambda i: (0,)),
            pl.BlockSpec((16, 3), lambda i: (0, 0)),
            pl.BlockSpec((3,), lambda i: (0,)),
        ],
        out_specs=pl.BlockSpec((TC_BLK, 3), lambda i: (i, 0)),
        out_shape=jax.ShapeDtypeStruct((NPAD, 3), jnp.float32),
    )(parts, W1, b1, W2, b2)
    return full[:N_NODES]


def kernel(x, hyperedge_index, W1, b1, W2, b2):
    node_idx = hyperedge_index[0]
    edge_idx = hyperedge_index[1]

    npad = NNZP - NNZ
    pad_g = (jnp.arange(npad, dtype=jnp.int32) * 997) % N_NODES
    pad_s = N_NODES + (jnp.arange(npad, dtype=jnp.int32) % N_DUMP)

    gA = jnp.concatenate([node_idx, pad_g])
    sA = jnp.concatenate([edge_idx, pad_s])
    gB = jnp.concatenate([edge_idx, pad_g])
    sB = jnp.concatenate([node_idx, pad_s])

    xp = jnp.concatenate(
        [x, jnp.ones((N_NODES, 1), jnp.float32),
         jnp.zeros((N_NODES, 4), jnp.float32)], axis=1)
    xp = jnp.concatenate([xp, jnp.zeros((N_DUMP, 8), jnp.float32)], axis=0)
    ztab = jnp.zeros((NPAD, 8), jnp.float32)

    v = _tc_combine(_sc_phase(xp, gA, sA, ztab))
    v = _tc_combine(_sc_phase(v, gB, sB, ztab))
    v = _tc_combine(_sc_phase(v, gA, sA, ztab))
    parts = _sc_phase(v, gB, sB, ztab)
    return _tc_final(parts, W1, b1, W2, b2)


# speedup vs baseline: 69.0721x; 4.9038x over previous
import functools

import jax
import jax.numpy as jnp
from jax import lax
from jax.experimental import pallas as pl
from jax.experimental.pallas import tpu as pltpu
from jax.experimental.pallas import tpu_sc as plsc

N_NODES = 100000
N_HEDGES = 100000
NNZ = 6400000

BATCH = 128
N_TILES = 32
UNROLL = 8
N_SLAB = 198
OUTER = N_SLAB // 6
N_SLOTS = 3
ROWS_PER_TILE = N_SLAB * UNROLL
ROWS = ROWS_PER_TILE * N_TILES
NNZP = ROWS * BATCH
N_DUMP = 96
NPAD = N_NODES + N_DUMP
ZR = NPAD // 16


def _sc_phase(vtab, gidx, sidx, ztab):
    mesh = plsc.VectorSubcoreMesh(core_axis_name="c", subcore_axis_name="s")

    @functools.partial(
        pl.kernel,
        out_type=jax.ShapeDtypeStruct((2, NPAD, 8), jnp.float32),
        mesh=mesh,
        scratch_types=[
            pltpu.VMEM((N_SLOTS, UNROLL, BATCH), jnp.int32),
            pltpu.VMEM((N_SLOTS, 2, UNROLL, BATCH), jnp.int32),
            pltpu.VMEM((N_SLOTS, UNROLL, BATCH, 8), jnp.float32),
            pltpu.VMEM_SHARED((NPAD, 8), jnp.float32),
            pltpu.SemaphoreType.DMA((N_SLOTS,)),
            pltpu.SemaphoreType.DMA((N_SLOTS, 2)),
            pltpu.SemaphoreType.DMA((N_SLOTS,)),
            pltpu.SemaphoreType.DMA((N_SLOTS,)),
        ],
        compiler_params=pltpu.CompilerParams(use_tc_tiling_on_sc=False),
    )
    def phase(vtab_hbm, gidx_hbm, sidx_hbm, ztab_hbm, out_hbm,
              gbuf, sbuf, rbuf, acc, sem_gi, sem_si, sem_g, sem_sc):
        cid = lax.axis_index("c")
        sid = lax.axis_index("s")
        wid = sid * 2 + cid

        pltpu.sync_copy(ztab_hbm.at[pl.ds(sid * ZR, ZR), :],
                        acc.at[pl.ds(sid * ZR, ZR), :])
        plsc.subcore_barrier()

        base = wid * N_SLAB

        def idx_copy(j, b, p):
            row = (base + j) * UNROLL
            pltpu.async_copy(gidx_hbm.at[pl.ds(row, UNROLL), :],
                             gbuf.at[b], sem_gi.at[b])
            pltpu.async_copy(sidx_hbm.at[pl.ds(row, UNROLL), :],
                             sbuf.at[b, p], sem_si.at[b, p])

        for b in range(N_SLOTS):
            idx_copy(jnp.int32(b), b, 0)

        def visit(k, u, j):
            b, p = u % 3, u // 3

            def drain_sc():
                for kk in range(UNROLL):
                    pltpu.make_async_copy(
                        rbuf.at[b, kk], acc.at[sbuf.at[b, 1 - p, kk]],
                        sem_sc.at[b]).wait()

            if u < 3:
                pl.when(k > 0)(drain_sc)
            else:
                drain_sc()

            row = (base + j) * UNROLL
            pltpu.make_async_copy(gidx_hbm.at[pl.ds(row, UNROLL), :],
                                  gbuf.at[b], sem_gi.at[b]).wait()
            pltpu.make_async_copy(sidx_hbm.at[pl.ds(row, UNROLL), :],
                                  sbuf.at[b, p], sem_si.at[b, p]).wait()

            for kk in range(UNROLL):
                pltpu.async_copy(vtab_hbm.at[gbuf.at[b, kk]],
                                 rbuf.at[b, kk], sem_g.at[b])
            for kk in range(UNROLL):
                pltpu.make_async_copy(vtab_hbm.at[gbuf.at[b, kk]],
                                      rbuf.at[b, kk], sem_g.at[b]).wait()

            for kk in range(UNROLL):
                pltpu.async_copy(rbuf.at[b, kk], acc.at[sbuf.at[b, p, kk]],
                                 sem_sc.at[b], add=True)

            pl.when(j + 3 < N_SLAB)(lambda: idx_copy(j + 3, b, 1 - p))

        def outer(k, carry):
            for u in range(6):
                visit(k, u, k * 6 + u)
            return carry

        lax.fori_loop(0, OUTER, outer, 0)

        for b in range(N_SLOTS):
            for kk in range(UNROLL):
                pltpu.make_async_copy(rbuf.at[b, kk],
                                      acc.at[sbuf.at[b, 1, kk]],
                                      sem_sc.at[b]).wait()
        plsc.subcore_barrier()

        pltpu.sync_copy(acc.at[pl.ds(sid * ZR, ZR), :],
                        out_hbm.at[cid, pl.ds(sid * ZR, ZR), :])

    return phase(vtab, gidx, sidx, ztab)


TC_BLK = 3128
TC_GRID = NPAD // TC_BLK


def _tc_combine(parts):
    def body(p_ref, o_ref):
        p = p_ref[0] + p_ref[1]
        cnt = p[:, 3:4]
        o_ref[...] = p * jnp.where(cnt > 0, 1.0 / cnt, 0.0)

    return pl.pallas_call(
        body,
        grid=(TC_GRID,),
        in_specs=[pl.BlockSpec((2, TC_BLK, 8), lambda i: (0, i, 0))],
        out_specs=pl.BlockSpec((TC_BLK, 8), lambda i: (i, 0)),
        out_shape=jax.ShapeDtypeStruct((NPAD, 8), jnp.float32),
    )(parts)


def _tc_final(parts, W1, b1, W2, b2):
    def body(p_ref, w1_ref, b1_ref, w2_ref, b2_ref, o_ref):
        p = p_ref[0] + p_ref[1]
        cnt = p[:, 3:4]
        scale = jnp.where(cnt > 0, 1.0 / cnt, 0.0)
        z = p[:, :3] * scale
        d = jnp.where(cnt > 0, 1.0, 0.0)
        t = (jnp.dot(z, w1_ref[...], preferred_element_type=jnp.float32)
             + d * b1_ref[...][None, :])
        o_ref[...] = (jnp.dot(t, w2_ref[...], preferred_element_type=jnp.float32)
                      + b2_ref[...][None, :])

    full = pl.pallas_call(
        body,
        grid=(TC_GRID,),
        in_specs=[
            pl.BlockSpec((2, TC_BLK, 8), lambda i: (0, i, 0)),
            pl.BlockSpec((3, 16), lambda i: (0, 0)),
            pl.BlockSpec((16,), lambda i: (0,)),
            pl.BlockSpec((16, 3), lambda i: (0, 0)),
            pl.BlockSpec((3,), lambda i: (0,)),
        ],
        out_specs=pl.BlockSpec((TC_BLK, 3), lambda i: (i, 0)),
        out_shape=jax.ShapeDtypeStruct((NPAD, 3), jnp.float32),
    )(parts, W1, b1, W2, b2)
    return full[:N_NODES]


def kernel(x, hyperedge_index, W1, b1, W2, b2):
    node_idx = hyperedge_index[0]
    edge_idx = hyperedge_index[1]

    npad = NNZP - NNZ
    pad_g = (jnp.arange(npad, dtype=jnp.int32) * 997) % N_NODES
    pad_s = N_NODES + (jnp.arange(npad, dtype=jnp.int32) % N_DUMP)

    gA = jnp.concatenate([node_idx, pad_g]).reshape(ROWS, BATCH)
    sA = jnp.concatenate([edge_idx, pad_s]).reshape(ROWS, BATCH)
    gB = jnp.concatenate([edge_idx, pad_g]).reshape(ROWS, BATCH)
    sB = jnp.concatenate([node_idx, pad_s]).reshape(ROWS, BATCH)

    xp = jnp.concatenate(
        [x, jnp.ones((N_NODES, 1), jnp.float32),
         jnp.zeros((N_NODES, 4), jnp.float32)], axis=1)
    xp = jnp.concatenate([xp, jnp.zeros((N_DUMP, 8), jnp.float32)], axis=0)
    ztab = jnp.zeros((NPAD, 8), jnp.float32)

    v = _tc_combine(_sc_phase(xp, gA, sA, ztab))
    v = _tc_combine(_sc_phase(v, gB, sB, ztab))
    v = _tc_combine(_sc_phase(v, gA, sA, ztab))
    parts = _sc_phase(v, gB, sB, ztab)
    return _tc_final(parts, W1, b1, W2, b2)
